# baseline (speedup 1.0000x reference)
import functools
import math

import jax
import jax.numpy as jnp
from jax import lax
from jax.experimental import pallas as pl
from jax.experimental.pallas import tpu as pltpu
from jax.experimental.pallas import tpu_sc as plsc

_NUM_CORES = 2
_NUM_SUBCORES = 16
_NUM_WORKERS = _NUM_CORES * _NUM_SUBCORES
_LANES = 16

_CHUNK = 16
_NROW = 5
_NPOS = 2
_AHEAD = 2


def _build(B, S, V, D):
    s_per_w = S // _NUM_WORKERS
    n_chunks = s_per_w // _CHUNK
    n_units = n_chunks * B
    scale = jnp.float32(math.sqrt(D))
    vecs_per_row = D // _LANES

    mesh = plsc.VectorSubcoreMesh(core_axis_name="c", subcore_axis_name="s")

    row_bufs = [pltpu.VMEM((_CHUNK, D), jnp.float32) for _ in range(_NROW)]
    pos_bufs = [pltpu.VMEM((_CHUNK, D), jnp.float32) for _ in range(_NPOS)]
    sems = [pltpu.SemaphoreType.DMA for _ in range(_NROW + _NROW + _NPOS)]

    @functools.partial(
        pl.kernel,
        mesh=mesh,
        out_type=jax.ShapeDtypeStruct((B, S, D), jnp.float32),
        scratch_types=[pltpu.VMEM((B, s_per_w), jnp.int32)]
        + row_bufs + pos_bufs + sems,
    )
    def embed(tok_hbm, ttab_hbm, ptab_hbm, out_hbm, idx_v, *bufs):
        row_v = bufs[:_NROW]
        pos_v = bufs[_NROW:_NROW + _NPOS]
        g_sem = bufs[_NROW + _NPOS:2 * _NROW + _NPOS]
        w_sem = bufs[2 * _NROW + _NPOS:3 * _NROW + _NPOS]
        p_sem = bufs[3 * _NROW + _NPOS:]

        wid = lax.axis_index("s") * _NUM_CORES + lax.axis_index("c")
        s_base = wid * s_per_w

        pltpu.sync_copy(tok_hbm.at[:, pl.ds(s_base, s_per_w)], idx_v)

        def pos_load(j):
            return pltpu.async_copy(
                ptab_hbm.at[pl.ds(s_base + j * _CHUNK, _CHUNK)],
                pos_v[j % _NPOS], p_sem[j % _NPOS])

        def gather(u):
            j, b = divmod(u, B)
            idx = idx_v.at[b, pl.ds(j * _CHUNK, _CHUNK)]
            return pltpu.async_copy(ttab_hbm.at[idx], row_v[u % _NROW],
                                    g_sem[u % _NROW])

        def write(u):
            j, b = divmod(u, B)
            return pltpu.async_copy(
                row_v[u % _NROW],
                out_hbm.at[b, pl.ds(s_base + j * _CHUNK, _CHUNK)],
                w_sem[u % _NROW])

        pos_h = {j: pos_load(j) for j in range(min(_NPOS, n_chunks))}
        g_h = {u: gather(u) for u in range(min(_AHEAD + 1, n_units))}
        w_h = {}

        for u in range(n_units):
            j, b = divmod(u, B)
            nu = u + _AHEAD
            if nu < n_units and nu > _AHEAD:
                if nu >= _NROW:
                    w_h[nu - _NROW].wait()
                g_h[nu] = gather(nu)
            g_h[u].wait()
            if b == 0:
                pos_h[j].wait()

            rv, pv = row_v[u % _NROW], pos_v[j % _NPOS]

            def row_body(r, _, rv=rv, pv=pv):
                def cg_body(cg, _):
                    base = cg * (_LANES * 16)
                    for c in range(16):
                        sl = pl.ds(base + c * _LANES, _LANES)
                        rv[r, sl] = (rv[r, sl] + pv[r, sl]) * scale
                    return _

                lax.fori_loop(0, vecs_per_row // 16, cg_body, 0)
                return _

            lax.fori_loop(0, _CHUNK, row_body, 0)
            w_h[u] = write(u)
            if b == B - 1 and j + _NPOS < n_chunks:
                pos_h[j + _NPOS] = pos_load(j + _NPOS)

        for u in range(max(0, n_units - _NROW), n_units):
            w_h[u].wait()

    return embed


def kernel(tokens, token_table, pos_table):
    B, S = tokens.shape
    V, D = token_table.shape
    embed = _build(B, S, V, D)
    return embed(tokens.astype(jnp.int32), token_table, pos_table)


# speedup vs baseline: 2.6089x; 2.6089x over previous
import functools
import math

import jax
import jax.numpy as jnp
from jax import lax
from jax.experimental import pallas as pl
from jax.experimental.pallas import tpu as pltpu
from jax.experimental.pallas import tpu_sc as plsc

_NUM_CORES = 2
_NUM_SUBCORES = 16
_NUM_WORKERS = _NUM_CORES * _NUM_SUBCORES
_LANES = 16

_CHUNK = 16
_NROW = 5
_NPOS = 2
_AHEAD = 3


def _build(B, S, V, D):
    s_per_w = S // _NUM_WORKERS
    n_chunks = s_per_w // _CHUNK
    n_units = n_chunks * B
    scale = jnp.float32(math.sqrt(D))
    vecs_per_row = D // _LANES

    mesh = plsc.VectorSubcoreMesh(core_axis_name="c", subcore_axis_name="s")

    row_bufs = [pltpu.VMEM((_CHUNK, D), jnp.float32) for _ in range(_NROW)]
    pos_bufs = [pltpu.VMEM((_CHUNK, D), jnp.float32) for _ in range(_NPOS)]
    sems = [pltpu.SemaphoreType.DMA for _ in range(_NROW + _NROW + _NPOS)]

    @functools.partial(
        pl.kernel,
        mesh=mesh,
        out_type=jax.ShapeDtypeStruct((B, S, D), jnp.float32),
        scratch_types=[pltpu.VMEM((B, s_per_w), jnp.int32)]
        + row_bufs + pos_bufs + sems,
    )
    def embed(tok_hbm, ttab_hbm, ptab_hbm, out_hbm, idx_v, *bufs):
        row_v = bufs[:_NROW]
        pos_v = bufs[_NROW:_NROW + _NPOS]
        g_sem = bufs[_NROW + _NPOS:2 * _NROW + _NPOS]
        w_sem = bufs[2 * _NROW + _NPOS:3 * _NROW + _NPOS]
        p_sem = bufs[3 * _NROW + _NPOS:]

        wid = lax.axis_index("s") * _NUM_CORES + lax.axis_index("c")
        s_base = wid * s_per_w

        pltpu.sync_copy(tok_hbm.at[:, pl.ds(s_base, s_per_w)], idx_v)

        def pos_load(j):
            return pltpu.async_copy(
                ptab_hbm.at[pl.ds(s_base + j * _CHUNK, _CHUNK)],
                pos_v[j % _NPOS], p_sem[j % _NPOS])

        def gather(u):
            j, b = divmod(u, B)
            idx = idx_v.at[b, pl.ds(j * _CHUNK, _CHUNK)]
            return pltpu.async_copy(ttab_hbm.at[idx], row_v[u % _NROW],
                                    g_sem[u % _NROW])

        def write(u):
            j, b = divmod(u, B)
            return pltpu.async_copy(
                row_v[u % _NROW],
                out_hbm.at[b, pl.ds(s_base + j * _CHUNK, _CHUNK)],
                w_sem[u % _NROW])

        pos_h = {j: pos_load(j) for j in range(min(_NPOS, n_chunks))}
        g_h = {u: gather(u) for u in range(min(_AHEAD + 1, n_units))}
        w_h = {}

        for u in range(n_units):
            j, b = divmod(u, B)
            nu = u + _AHEAD
            if nu < n_units and nu > _AHEAD:
                if nu >= _NROW:
                    w_h[nu - _NROW].wait()
                g_h[nu] = gather(nu)
            g_h[u].wait()
            if b == 0:
                pos_h[j].wait()

            rv, pv = row_v[u % _NROW], pos_v[j % _NPOS]

            def row_body(r, _, rv=rv, pv=pv):
                for c in range(vecs_per_row):
                    sl = pl.ds(c * _LANES, _LANES)
                    rv[r, sl] = (rv[r, sl] + pv[r, sl]) * scale
                return _

            lax.fori_loop(0, _CHUNK, row_body, 0)
            w_h[u] = write(u)
            if b == B - 1 and j + _NPOS < n_chunks:
                pos_h[j + _NPOS] = pos_load(j + _NPOS)

        for u in range(max(0, n_units - _NROW), n_units):
            w_h[u].wait()

    return embed


def kernel(tokens, token_table, pos_table):
    B, S = tokens.shape
    V, D = token_table.shape
    embed = _build(B, S, V, D)
    return embed(tokens.astype(jnp.int32), token_table, pos_table)


# speedup vs baseline: 2.6166x; 1.0030x over previous
import functools
import math

import jax
import jax.numpy as jnp
from jax import lax
from jax.experimental import pallas as pl
from jax.experimental.pallas import tpu as pltpu
from jax.experimental.pallas import tpu_sc as plsc

_NUM_CORES = 2
_NUM_SUBCORES = 16
_NUM_WORKERS = _NUM_CORES * _NUM_SUBCORES
_LANES = 16

_CHUNK = 16
_NROW = 5
_NPOS = 2
_AHEAD = 2


def _build(B, S, V, D):
    s_per_w = S // _NUM_WORKERS
    n_chunks = s_per_w // _CHUNK
    n_units = n_chunks * B
    scale = jnp.float32(math.sqrt(D))
    vecs_per_row = D // _LANES

    mesh = plsc.VectorSubcoreMesh(core_axis_name="c", subcore_axis_name="s")

    row_bufs = [pltpu.VMEM((_CHUNK, D), jnp.float32) for _ in range(_NROW)]
    pos_bufs = [pltpu.VMEM((_CHUNK, D), jnp.float32) for _ in range(_NPOS)]
    sems = [pltpu.SemaphoreType.DMA for _ in range(_NROW + _NROW + _NPOS)]

    @functools.partial(
        pl.kernel,
        mesh=mesh,
        out_type=jax.ShapeDtypeStruct((B, S, D), jnp.float32),
        scratch_types=[pltpu.VMEM((B, s_per_w), jnp.int32)]
        + row_bufs + pos_bufs + sems,
    )
    def embed(tok_hbm, ttab_hbm, ptab_hbm, out_hbm, idx_v, *bufs):
        row_v = bufs[:_NROW]
        pos_v = bufs[_NROW:_NROW + _NPOS]
        g_sem = bufs[_NROW + _NPOS:2 * _NROW + _NPOS]
        w_sem = bufs[2 * _NROW + _NPOS:3 * _NROW + _NPOS]
        p_sem = bufs[3 * _NROW + _NPOS:]

        wid = lax.axis_index("s") * _NUM_CORES + lax.axis_index("c")
        s_base = wid * s_per_w

        pltpu.sync_copy(tok_hbm.at[:, pl.ds(s_base, s_per_w)], idx_v)

        def pos_load(j):
            return pltpu.async_copy(
                ptab_hbm.at[pl.ds(s_base + j * _CHUNK, _CHUNK)],
                pos_v[j % _NPOS], p_sem[j % _NPOS])

        def gather(u):
            j, b = divmod(u, B)
            idx = idx_v.at[b, pl.ds(j * _CHUNK, _CHUNK)]
            return pltpu.async_copy(ttab_hbm.at[idx], row_v[u % _NROW],
                                    g_sem[u % _NROW])

        def write(u):
            j, b = divmod(u, B)
            return pltpu.async_copy(
                row_v[u % _NROW],
                out_hbm.at[b, pl.ds(s_base + j * _CHUNK, _CHUNK)],
                w_sem[u % _NROW])

        pos_h = {j: pos_load(j) for j in range(min(_NPOS, n_chunks))}
        g_h = {u: gather(u) for u in range(min(_AHEAD + 1, n_units))}
        w_h = {}

        for u in range(n_units):
            j, b = divmod(u, B)
            nu = u + _AHEAD
            if nu < n_units and nu > _AHEAD:
                if nu >= _NROW:
                    w_h[nu - _NROW].wait()
                g_h[nu] = gather(nu)
            g_h[u].wait()
            if b == 0:
                pos_h[j].wait()

            rv, pv = row_v[u % _NROW], pos_v[j % _NPOS]

            def row_body(r, _, rv=rv, pv=pv):
                for c in range(vecs_per_row):
                    sl = pl.ds(c * _LANES, _LANES)
                    rv[r, sl] = (rv[r, sl] + pv[r, sl]) * scale
                return _

            lax.fori_loop(0, _CHUNK, row_body, 0)
            w_h[u] = write(u)
            if b == B - 1 and j + _NPOS < n_chunks:
                pos_h[j + _NPOS] = pos_load(j + _NPOS)

        for u in range(max(0, n_units - _NROW), n_units):
            w_h[u].wait()

    return embed


def kernel(tokens, token_table, pos_table):
    B, S = tokens.shape
    V, D = token_table.shape
    embed = _build(B, S, V, D)
    return embed(tokens.astype(jnp.int32), token_table, pos_table)
